# baseline (speedup 1.0000x reference)
import functools

import jax
import jax.numpy as jnp
from jax import lax
from jax.experimental import pallas as pl
from jax.experimental.pallas import tpu as pltpu
from jax.experimental.pallas import tpu_sc as plsc

N = 10000
E = 320000
D = 128
NC = 2
NS = 16
NW = NC * NS
EPW = E // NW
C = 128
K = 80
KB = 16
NB = K // KB
C2 = 64
K2 = (K * C) // C2
KB2 = 32
NB2 = K2 // KB2
NRING = 5
SRCB = KB2 * C2
NPAD = 10240
RPT = NPAD // NS
NZC = RPT // C
TRASH = N

_mesh = plsc.VectorSubcoreMesh(core_axis_name="c", subcore_axis_name="s")


def _zero_rows(buf):
    def zrow(r, _):
        for cc in range(8):
            buf[r, pl.ds(cc * 16, 16)] = jnp.zeros((16,), jnp.float32)
        return 0

    lax.fori_loop(0, buf.shape[0], zrow, 0)


@functools.partial(
    pl.kernel,
    out_type=jax.ShapeDtypeStruct((NC * NPAD,), jnp.float32),
    mesh=_mesh,
    scratch_types=[
        pltpu.VMEM((KB, C), jnp.int32),
        pltpu.VMEM((C,), jnp.float32),
        pltpu.VMEM((RPT,), jnp.float32),
        pltpu.VMEM_SHARED((NPAD,), jnp.float32),
    ],
)
def _sc_deg(dst_hbm, out_hbm, dst_v, ones_v, stage_v, hist_sh):
    c = lax.axis_index("c")
    s = lax.axis_index("s")
    w = s * NC + c

    def fill(i, _):
        ones_v[pl.ds(i * 16, 16)] = jnp.ones((16,), jnp.float32)
        return 0

    lax.fori_loop(0, C // 16, fill, 0)

    def zfill(i, _):
        stage_v[pl.ds(i * 16, 16)] = jnp.zeros((16,), jnp.float32)
        return 0

    lax.fori_loop(0, RPT // 16, zfill, 0)
    pltpu.sync_copy(stage_v, hist_sh.at[pl.ds(s * RPT, RPT)])
    plsc.subcore_barrier()

    def step(j, _):
        pltpu.sync_copy(ones_v, hist_sh.at[dst_v.at[j]], add=True)
        return 0

    for ob in range(NB):
        pltpu.sync_copy(dst_hbm.at[w, pl.ds(ob * KB, KB)], dst_v)
        lax.fori_loop(0, KB, step, 0)
    plsc.subcore_barrier()

    pltpu.sync_copy(hist_sh.at[pl.ds(s * RPT, RPT)], stage_v)
    pltpu.sync_copy(stage_v, out_hbm.at[pl.ds(c * NPAD + s * RPT, RPT)])


@functools.partial(
    pl.kernel,
    out_type=jax.ShapeDtypeStruct((NC, NPAD, D), jnp.float32),
    mesh=_mesh,
    scratch_types=[
        pltpu.VMEM((2, SRCB), jnp.int32),
        pltpu.VMEM((KB2, C2), jnp.int32),
        [pltpu.VMEM((C2, D), jnp.float32) for _ in range(NRING)],
        pltpu.VMEM_SHARED((NPAD, D), jnp.float32),
        [pltpu.SemaphoreType.DMA for _ in range(NRING)],
    ],
)
def _sc_agg(g_hbm, src_hbm, dst_hbm, out_hbm, src_v, dst_v, ring, agg_sh, sems):
    c = lax.axis_index("c")
    s = lax.axis_index("s")
    w = s * NC + c

    pltpu.sync_copy(src_hbm.at[w, pl.ds(0, SRCB)], src_v.at[0])

    def gsl(j):
        return g_hbm.at[src_v.at[(j // KB2) % 2, pl.ds((j % KB2) * C2, C2)]]

    for r in range(1, NRING):
        pltpu.async_copy(gsl(r), ring[r], sems[r])

    _zero_rows(ring[0])
    for k in range(RPT // C2):
        pltpu.sync_copy(ring[0], agg_sh.at[pl.ds(s * RPT + k * C2, C2)])
    plsc.subcore_barrier()
    pltpu.async_copy(gsl(0), ring[0], sems[0])

    def step(j, _):
        for r in range(NRING):
            @pl.when(j % NRING == r)
            def _():
                pltpu.make_async_copy(gsl(j), ring[r], sems[r]).wait()
                pltpu.sync_copy(ring[r], agg_sh.at[dst_v.at[j % KB2]], add=True)

                @pl.when(j + NRING < K2)
                def _():
                    pltpu.async_copy(gsl(j + NRING), ring[r], sems[r])

        return 0

    for ob in range(NB2):
        if ob + 1 < NB2:
            pltpu.sync_copy(src_hbm.at[w, pl.ds((ob + 1) * SRCB, SRCB)],
                            src_v.at[(ob + 1) % 2])
        pltpu.sync_copy(dst_hbm.at[w, pl.ds(ob * KB2, KB2)], dst_v)
        lax.fori_loop(ob * KB2, (ob + 1) * KB2, step, 0)
    plsc.subcore_barrier()

    off = pl.ds(s * RPT, RPT)
    pltpu.sync_copy(agg_sh.at[off], out_hbm.at[c, off])


_BLK = 1000
_GRID = N // _BLK


def _dinv_of(deg_ref):
    return lax.rsqrt(deg_ref[0] + deg_ref[1] + 1.0)


def _tc1_body(x_ref, w_ref, deg_ref, g_ref):
    dinv = _dinv_of(deg_ref)
    h = jnp.dot(x_ref[...], w_ref[...], preferred_element_type=jnp.float32)
    g_ref[...] = h * dinv


def _tc1(x, W1, deg2):
    return pl.pallas_call(
        _tc1_body,
        grid=(_GRID,),
        in_specs=[
            pl.BlockSpec((_BLK, D), lambda i: (i, 0)),
            pl.BlockSpec((D, D), lambda i: (0, 0)),
            pl.BlockSpec((NC, _BLK, 1), lambda i: (0, i, 0)),
        ],
        out_specs=pl.BlockSpec((_BLK, D), lambda i: (i, 0)),
        out_shape=jax.ShapeDtypeStruct((N, D), jnp.float32),
    )(x, W1, deg2)


def _tc2_body(p_ref, g_ref, deg_ref, b_ref, w_ref, o_ref):
    dinv = _dinv_of(deg_ref)
    ssum = p_ref[0] + p_ref[1] + g_ref[...]
    out1 = jnp.maximum(ssum * dinv + b_ref[...][None, :], 0.0)
    o_ref[...] = jnp.dot(out1, w_ref[...],
                         preferred_element_type=jnp.float32) * dinv


def _tc2(p, g1, deg2, b1, W2):
    return pl.pallas_call(
        _tc2_body,
        grid=(_GRID,),
        in_specs=[
            pl.BlockSpec((NC, _BLK, D), lambda i: (0, i, 0)),
            pl.BlockSpec((_BLK, D), lambda i: (i, 0)),
            pl.BlockSpec((NC, _BLK, 1), lambda i: (0, i, 0)),
            pl.BlockSpec((D,), lambda i: (0,)),
            pl.BlockSpec((D, D), lambda i: (0, 0)),
        ],
        out_specs=pl.BlockSpec((_BLK, D), lambda i: (i, 0)),
        out_shape=jax.ShapeDtypeStruct((N, D), jnp.float32),
    )(p, g1, deg2, b1, W2)


def _tc3_body(q_ref, g_ref, deg_ref, b_ref, wc_ref, bc_ref, o_ref):
    dinv = _dinv_of(deg_ref)
    ssum = q_ref[0] + q_ref[1] + g_ref[...]
    out2 = jnp.maximum(ssum * dinv + b_ref[...][None, :], 0.0)
    o_ref[...] = (jnp.dot(out2, wc_ref[...], preferred_element_type=jnp.float32)
                  + bc_ref[...][None, :])


def _tc3(q, g2, deg2, b2, Wc, bc):
    return pl.pallas_call(
        _tc3_body,
        grid=(_GRID,),
        in_specs=[
            pl.BlockSpec((NC, _BLK, D), lambda i: (0, i, 0)),
            pl.BlockSpec((_BLK, D), lambda i: (i, 0)),
            pl.BlockSpec((NC, _BLK, 1), lambda i: (0, i, 0)),
            pl.BlockSpec((D,), lambda i: (0,)),
            pl.BlockSpec((D, 64), lambda i: (0, 0)),
            pl.BlockSpec((64,), lambda i: (0,)),
        ],
        out_specs=pl.BlockSpec((_BLK, 64), lambda i: (i, 0)),
        out_shape=jax.ShapeDtypeStruct((N, 64), jnp.float32),
    )(q, g2, deg2, b2, Wc, bc)


def kernel(x, edge_index, W1, b1, W2, b2, Wc, bc):
    ei = edge_index.astype(jnp.int32)
    pad = K * C - EPW
    src3 = jnp.pad(ei[0].reshape(NW, EPW), ((0, 0), (0, pad)),
                   constant_values=0).reshape(NW, K, C)
    dst3 = jnp.pad(ei[1].reshape(NW, EPW), ((0, 0), (0, pad)),
                   constant_values=TRASH).reshape(NW, K, C)

    deg2 = _sc_deg(dst3).reshape(NC, NPAD, 1)
    g1 = _tc1(x, W1, deg2)
    srcf = src3.reshape(NW, K * C)
    dst64 = dst3.reshape(NW, K2, C2)
    p = _sc_agg(g1, srcf, dst64)
    g2 = _tc2(p, g1, deg2, b1, W2)
    q = _sc_agg(g2, srcf, dst64)
    out = _tc3(q, g2, deg2, b2, Wc, bc)
    return (out, jnp.asarray(0.0, dtype=jnp.float32))


# speedup vs baseline: 1.0034x; 1.0034x over previous
import functools

import jax
import jax.numpy as jnp
from jax import lax
from jax.experimental import pallas as pl
from jax.experimental.pallas import tpu as pltpu
from jax.experimental.pallas import tpu_sc as plsc

N = 10000
E = 320000
D = 128
NC = 2
NS = 16
NW = NC * NS
EPW = E // NW
C = 128
K = 80
KB = 16
NB = K // KB
C2 = 64
K2 = (K * C) // C2
KB2 = 32
NB2 = K2 // KB2
NRING = 4
NPAD = 10240
RPT = NPAD // NS
NZC = RPT // C
TRASH = N

_mesh = plsc.VectorSubcoreMesh(core_axis_name="c", subcore_axis_name="s")


def _zero_rows(buf):
    def zrow(r, _):
        for cc in range(8):
            buf[r, pl.ds(cc * 16, 16)] = jnp.zeros((16,), jnp.float32)
        return 0

    lax.fori_loop(0, buf.shape[0], zrow, 0)


@functools.partial(
    pl.kernel,
    out_type=jax.ShapeDtypeStruct((NC * NPAD,), jnp.float32),
    mesh=_mesh,
    scratch_types=[
        pltpu.VMEM((KB, C), jnp.int32),
        pltpu.VMEM((C,), jnp.float32),
        pltpu.VMEM((RPT,), jnp.float32),
        pltpu.VMEM_SHARED((NPAD,), jnp.float32),
    ],
)
def _sc_deg(dst_hbm, out_hbm, dst_v, ones_v, stage_v, hist_sh):
    c = lax.axis_index("c")
    s = lax.axis_index("s")
    w = s * NC + c

    def fill(i, _):
        ones_v[pl.ds(i * 16, 16)] = jnp.ones((16,), jnp.float32)
        return 0

    lax.fori_loop(0, C // 16, fill, 0)

    def zfill(i, _):
        stage_v[pl.ds(i * 16, 16)] = jnp.zeros((16,), jnp.float32)
        return 0

    lax.fori_loop(0, RPT // 16, zfill, 0)
    pltpu.sync_copy(stage_v, hist_sh.at[pl.ds(s * RPT, RPT)])
    plsc.subcore_barrier()

    def step(j, _):
        pltpu.sync_copy(ones_v, hist_sh.at[dst_v.at[j]], add=True)
        return 0

    for ob in range(NB):
        pltpu.sync_copy(dst_hbm.at[w, pl.ds(ob * KB, KB)], dst_v)
        lax.fori_loop(0, KB, step, 0)
    plsc.subcore_barrier()

    pltpu.sync_copy(hist_sh.at[pl.ds(s * RPT, RPT)], stage_v)
    pltpu.sync_copy(stage_v, out_hbm.at[pl.ds(c * NPAD + s * RPT, RPT)])


@functools.partial(
    pl.kernel,
    out_type=jax.ShapeDtypeStruct((NC, NPAD, D), jnp.float32),
    mesh=_mesh,
    scratch_types=[
        pltpu.VMEM((K * C,), jnp.int32),
        pltpu.VMEM((KB2, C2), jnp.int32),
        [pltpu.VMEM((C2, D), jnp.float32) for _ in range(NRING)],
        pltpu.VMEM_SHARED((NPAD, D), jnp.float32),
        [pltpu.SemaphoreType.DMA for _ in range(NRING)],
    ],
)
def _sc_agg(g_hbm, src_hbm, dst_hbm, out_hbm, src_v, dst_v, ring, agg_sh, sems):
    c = lax.axis_index("c")
    s = lax.axis_index("s")
    w = s * NC + c

    pltpu.sync_copy(src_hbm.at[w], src_v)

    def gsl(j):
        return g_hbm.at[src_v.at[pl.ds(j * C2, C2)]]

    for r in range(1, NRING):
        pltpu.async_copy(gsl(r), ring[r], sems[r])

    _zero_rows(ring[0])
    for k in range(RPT // C2):
        pltpu.sync_copy(ring[0], agg_sh.at[pl.ds(s * RPT + k * C2, C2)])
    plsc.subcore_barrier()
    pltpu.async_copy(gsl(0), ring[0], sems[0])

    def step(j, _):
        for r in range(NRING):
            @pl.when(j % NRING == r)
            def _():
                pltpu.make_async_copy(gsl(j), ring[r], sems[r]).wait()
                pltpu.sync_copy(ring[r], agg_sh.at[dst_v.at[j % KB2]], add=True)

                @pl.when(j + NRING < K2)
                def _():
                    pltpu.async_copy(gsl(j + NRING), ring[r], sems[r])

        return 0

    for ob in range(NB2):
        pltpu.sync_copy(dst_hbm.at[w, pl.ds(ob * KB2, KB2)], dst_v)
        lax.fori_loop(ob * KB2, (ob + 1) * KB2, step, 0)
    plsc.subcore_barrier()

    off = pl.ds(s * RPT, RPT)
    pltpu.sync_copy(agg_sh.at[off], out_hbm.at[c, off])


_BLK = 1000
_GRID = N // _BLK


def _dinv_of(deg_ref):
    return lax.rsqrt(deg_ref[0] + deg_ref[1] + 1.0)


def _tc1a_body(x_ref, w_ref, h_ref):
    h_ref[...] = jnp.dot(x_ref[...], w_ref[...],
                         preferred_element_type=jnp.float32)


def _tc1a(x, W1):
    return pl.pallas_call(
        _tc1a_body,
        grid=(_GRID,),
        in_specs=[
            pl.BlockSpec((_BLK, D), lambda i: (i, 0)),
            pl.BlockSpec((D, D), lambda i: (0, 0)),
        ],
        out_specs=pl.BlockSpec((_BLK, D), lambda i: (i, 0)),
        out_shape=jax.ShapeDtypeStruct((N, D), jnp.float32),
    )(x, W1)


def _tc1b_body(h_ref, deg_ref, g_ref):
    g_ref[...] = h_ref[...] * _dinv_of(deg_ref)


def _tc1b(h, deg2):
    return pl.pallas_call(
        _tc1b_body,
        grid=(_GRID,),
        in_specs=[
            pl.BlockSpec((_BLK, D), lambda i: (i, 0)),
            pl.BlockSpec((NC, _BLK, 1), lambda i: (0, i, 0)),
        ],
        out_specs=pl.BlockSpec((_BLK, D), lambda i: (i, 0)),
        out_shape=jax.ShapeDtypeStruct((N, D), jnp.float32),
    )(h, deg2)


def _tc2_body(p_ref, g_ref, deg_ref, b_ref, w_ref, o_ref):
    dinv = _dinv_of(deg_ref)
    ssum = p_ref[0] + p_ref[1] + g_ref[...]
    out1 = jnp.maximum(ssum * dinv + b_ref[...][None, :], 0.0)
    o_ref[...] = jnp.dot(out1, w_ref[...],
                         preferred_element_type=jnp.float32) * dinv


def _tc2(p, g1, deg2, b1, W2):
    return pl.pallas_call(
        _tc2_body,
        grid=(_GRID,),
        in_specs=[
            pl.BlockSpec((NC, _BLK, D), lambda i: (0, i, 0)),
            pl.BlockSpec((_BLK, D), lambda i: (i, 0)),
            pl.BlockSpec((NC, _BLK, 1), lambda i: (0, i, 0)),
            pl.BlockSpec((D,), lambda i: (0,)),
            pl.BlockSpec((D, D), lambda i: (0, 0)),
        ],
        out_specs=pl.BlockSpec((_BLK, D), lambda i: (i, 0)),
        out_shape=jax.ShapeDtypeStruct((N, D), jnp.float32),
    )(p, g1, deg2, b1, W2)


def _tc3_body(q_ref, g_ref, deg_ref, b_ref, wc_ref, bc_ref, o_ref):
    dinv = _dinv_of(deg_ref)
    ssum = q_ref[0] + q_ref[1] + g_ref[...]
    out2 = jnp.maximum(ssum * dinv + b_ref[...][None, :], 0.0)
    o_ref[...] = (jnp.dot(out2, wc_ref[...], preferred_element_type=jnp.float32)
                  + bc_ref[...][None, :])


def _tc3(q, g2, deg2, b2, Wc, bc):
    return pl.pallas_call(
        _tc3_body,
        grid=(_GRID,),
        in_specs=[
            pl.BlockSpec((NC, _BLK, D), lambda i: (0, i, 0)),
            pl.BlockSpec((_BLK, D), lambda i: (i, 0)),
            pl.BlockSpec((NC, _BLK, 1), lambda i: (0, i, 0)),
            pl.BlockSpec((D,), lambda i: (0,)),
            pl.BlockSpec((D, 64), lambda i: (0, 0)),
            pl.BlockSpec((64,), lambda i: (0,)),
        ],
        out_specs=pl.BlockSpec((_BLK, 64), lambda i: (i, 0)),
        out_shape=jax.ShapeDtypeStruct((N, 64), jnp.float32),
    )(q, g2, deg2, b2, Wc, bc)


def kernel(x, edge_index, W1, b1, W2, b2, Wc, bc):
    ei = edge_index.astype(jnp.int32)
    pad = K * C - EPW
    src3 = jnp.pad(ei[0].reshape(NW, EPW), ((0, 0), (0, pad)),
                   constant_values=0).reshape(NW, K, C)
    dst3 = jnp.pad(ei[1].reshape(NW, EPW), ((0, 0), (0, pad)),
                   constant_values=TRASH).reshape(NW, K, C)

    h1 = _tc1a(x, W1)
    deg2 = _sc_deg(dst3).reshape(NC, NPAD, 1)
    g1 = _tc1b(h1, deg2)
    srcf = src3.reshape(NW, K * C)
    dst64 = dst3.reshape(NW, K2, C2)
    p = _sc_agg(g1, srcf, dst64)
    g2 = _tc2(p, g1, deg2, b1, W2)
    q = _sc_agg(g2, srcf, dst64)
    out = _tc3(q, g2, deg2, b2, Wc, bc)
    return (out, jnp.asarray(0.0, dtype=jnp.float32))
